# baseline (speedup 1.0000x reference)
import functools

import jax
import jax.numpy as jnp
from jax import lax
from jax.experimental import pallas as pl
from jax.experimental.pallas import tpu as pltpu
from jax.experimental.pallas import tpu_sc as plsc

N = 50000
E = 800000
C = 64
HID = 128
DOUT = 100
DP = 112
DP2 = 128
DPW = 64

NC = 2
NS = 16
NW = NC * NS

A_CH = 256
A_PER_W = 2048
N_PAD = A_PER_W * NW
A_NCH = A_PER_W // A_CH

E_PAD = 802816
D_NCH = E_PAD // 128
D_PER_T = D_NCH // NS
D_IB = 56
N_ACC = 51200
ACC_PER_T = N_ACC // NS
SENTINEL = N

I_PER_W = E // NW
I_NCH = -(-I_PER_W // 128)


def _sc_mesh():
    return plsc.VectorSubcoreMesh(core_axis_name="c", subcore_axis_name="s")


_SC_PARAMS = pltpu.CompilerParams(use_tc_tiling_on_sc=False,
                                 needs_layout_passes=False)


def _emb_gather_body(lane_i, type_i, len_i, node_i,
                     small_t, node_t,
                     o_small, o_node,
                     ilane, itype, ilen, inode, ismall,
                     bsmall, bnode, sg0, sg1, sw0, sw1):
    wid = lax.axis_index("s") * NC + lax.axis_index("c")
    base = wid * A_PER_W
    pltpu.sync_copy(lane_i.at[pl.ds(base, A_PER_W)], ilane)
    pltpu.sync_copy(type_i.at[pl.ds(base, A_PER_W)], itype)
    pltpu.sync_copy(len_i.at[pl.ds(base, A_PER_W)], ilen)
    pltpu.sync_copy(node_i.at[pl.ds(base, A_PER_W)], inode)

    def fuse(i, carry):
        sl = pl.ds(i * 16, 16)
        ismall[sl] = (ilane[sl] * 20 + itype[sl]) * 100 + ilen[sl]
        return carry

    lax.fori_loop(0, A_PER_W // 16, fuse, 0)
    sg = (sg0, sg1)
    sw = (sw0, sw1)

    def issue(j, b):
        off = j * A_CH
        hb = pl.ds(b * A_CH, A_CH)
        pltpu.async_copy(small_t.at[ismall.at[pl.ds(off, A_CH)]],
                         bsmall.at[hb], sg[b])
        pltpu.async_copy(node_t.at[inode.at[pl.ds(off, A_CH)]],
                         bnode.at[hb], sg[b])

    def drain_g(b):
        hb = pl.ds(b * A_CH, A_CH)
        pltpu.make_async_copy(small_t.at[ismall.at[pl.ds(0, A_CH)]],
                              bsmall.at[hb], sg[b]).wait()
        pltpu.make_async_copy(node_t.at[inode.at[pl.ds(0, A_CH)]],
                              bnode.at[hb], sg[b]).wait()

    def put(j, b):
        gbase = base + j * A_CH
        hb = pl.ds(b * A_CH, A_CH)
        pltpu.async_copy(bsmall.at[hb], o_small.at[pl.ds(gbase, A_CH)], sw[b])
        pltpu.async_copy(bnode.at[hb], o_node.at[pl.ds(gbase, A_CH)], sw[b])

    def drain_w(b):
        hb = pl.ds(b * A_CH, A_CH)
        pltpu.make_async_copy(bsmall.at[hb], o_small.at[pl.ds(0, A_CH)],
                              sw[b]).wait()
        pltpu.make_async_copy(bnode.at[hb], o_node.at[pl.ds(0, A_CH)],
                              sw[b]).wait()

    issue(0, 0)
    issue(1, 1)

    def group(g, carry):
        for b in range(2):
            j = g * 2 + b
            drain_g(b)
            put(j, b)
            drain_w(b)
            issue(j + 2, b)
        return carry

    lax.fori_loop(0, (A_NCH - 2) // 2, group, 0)
    for b in range(2):
        j = A_NCH - 2 + b
        drain_g(b)
        put(j, b)
        drain_w(b)


def _emb_gather(lane_i, type_i, len_i, node_i, small_t, node_t):
    f32 = jnp.float32
    k = pl.kernel(
        _emb_gather_body,
        out_type=[
            jax.ShapeDtypeStruct((N_PAD, 64), f32),
            jax.ShapeDtypeStruct((N_PAD, 64), f32),
        ],
        mesh=_sc_mesh(),
        compiler_params=_SC_PARAMS,
        scratch_types=[
            pltpu.VMEM((A_PER_W,), jnp.int32),
            pltpu.VMEM((A_PER_W,), jnp.int32),
            pltpu.VMEM((A_PER_W,), jnp.int32),
            pltpu.VMEM((A_PER_W,), jnp.int32),
            pltpu.VMEM((A_PER_W,), jnp.int32),
            pltpu.VMEM((2 * A_CH, 64), f32),
            pltpu.VMEM((2 * A_CH, 64), f32),
            pltpu.SemaphoreType.DMA,
            pltpu.SemaphoreType.DMA,
            pltpu.SemaphoreType.DMA,
            pltpu.SemaphoreType.DMA,
        ],
    )
    return k(lane_i, type_i, len_i, node_i, small_t, node_t)


def _segsum_body(sa2, src2d, dst2d, zeros_h, out0, out1,
                 acc, sidx, didx, gbuf, zbuf, sem, sem2):
    c = lax.axis_index("c")
    s = lax.axis_index("s")
    arow = s * ACC_PER_T
    pltpu.sync_copy(zeros_h, zbuf)

    def zstep(w, carry):
        pltpu.sync_copy(zbuf, acc.at[pl.ds(arow + w * 128, 128)])
        return carry

    lax.fori_loop(0, ACC_PER_T // 128, zstep, 0)
    plsc.subcore_barrier()

    c32 = c.astype(jnp.int32)
    sg = (sem, sem2)

    def issue(jj, b):
        pltpu.async_copy(sa2.at[didx.at[jj]], gbuf.at[pl.ds(b * 128, 128)],
                         sg[b])

    def drain(b):
        pltpu.make_async_copy(sa2.at[didx.at[0]],
                              gbuf.at[pl.ds(b * 128, 128)], sg[b]).wait()

    def block(blk, carry):
        cbase = s * D_PER_T + blk * D_IB
        pltpu.sync_copy(src2d.at[pl.ds(cbase, D_IB)], sidx)
        pltpu.sync_copy(dst2d.at[pl.ds(cbase, D_IB)], didx)

        def tstep(jj, carry2):
            for k in range(8):
                sl = pl.ds(k * 16, 16)
                didx[jj, sl] = didx[jj, sl] * 2 + c32
            return carry2

        lax.fori_loop(0, D_IB, tstep, 0)
        issue(0, 0)
        issue(1, 1)

        def group(g, carry2):
            for b in range(2):
                jj = g * 2 + b
                drain(b)
                pltpu.sync_copy(gbuf.at[pl.ds(b * 128, 128)],
                                acc.at[sidx.at[jj]], add=True)
                issue(jj + 2, b)
            return carry2

        lax.fori_loop(0, (D_IB - 2) // 2, group, 0)
        for b in range(2):
            jj = D_IB - 2 + b
            drain(b)
            pltpu.sync_copy(gbuf.at[pl.ds(b * 128, 128)],
                            acc.at[sidx.at[jj]], add=True)
        return carry

    lax.fori_loop(0, D_PER_T // D_IB, block, 0)
    plsc.subcore_barrier()

    @pl.when(c == 0)
    def _():
        def wstep(w, carry):
            r = arow + w * 128
            pltpu.sync_copy(acc.at[pl.ds(r, 128)], zbuf)
            pltpu.sync_copy(zbuf, out0.at[pl.ds(r, 128)])
            return carry
        lax.fori_loop(0, ACC_PER_T // 128, wstep, 0)

    @pl.when(c == 1)
    def _():
        def wstep(w, carry):
            r = arow + w * 128
            pltpu.sync_copy(acc.at[pl.ds(r, 128)], zbuf)
            pltpu.sync_copy(zbuf, out1.at[pl.ds(r, 128)])
            return carry
        lax.fori_loop(0, ACC_PER_T // 128, wstep, 0)


def _segsum(sa2, src2d, dst2d, zeros_h):
    f32 = jnp.float32
    k = pl.kernel(
        _segsum_body,
        out_type=[
            jax.ShapeDtypeStruct((N_ACC, 32), f32),
            jax.ShapeDtypeStruct((N_ACC, 32), f32),
        ],
        mesh=_sc_mesh(),
        compiler_params=_SC_PARAMS,
        scratch_types=[
            pltpu.VMEM_SHARED((N_ACC, 32), f32),
            pltpu.VMEM((D_IB, 128), jnp.int32),
            pltpu.VMEM((D_IB, 128), jnp.int32),
            pltpu.VMEM((256, 32), f32),
            pltpu.VMEM((128, 32), f32),
            pltpu.SemaphoreType.DMA,
            pltpu.SemaphoreType.DMA,
        ],
    )
    return k(sa2, src2d, dst2d, zeros_h)


def _edge_dot_body(x_h, u_h, v_h, out, uia, via, ub, vb, ob,
                   sg0, sg1, sg2, sg3):
    wid = lax.axis_index("s") * NC + lax.axis_index("c")
    wbase = wid * I_PER_W
    pltpu.sync_copy(u_h.at[pl.ds(wbase, I_PER_W)], uia)
    pltpu.sync_copy(v_h.at[pl.ds(wbase, I_PER_W)], via)
    iota = lax.iota(jnp.int32, 16)
    sg = (sg0, sg1, sg2, sg3)
    himask = jnp.broadcast_to(jnp.int32(-65536), (16,))
    zero16 = jnp.zeros((16,), jnp.float32)

    def off_of(j):
        return jnp.minimum(j * 128, I_PER_W - 128)

    def issue(j, b):
        off = off_of(j)
        hb = pl.ds(b * 128, 128)
        pltpu.async_copy(x_h.at[uia.at[pl.ds(off, 128)]], ub.at[hb], sg[b])
        pltpu.async_copy(x_h.at[via.at[pl.ds(off, 128)]], vb.at[hb], sg[b])

    def drain(b):
        hb = pl.ds(b * 128, 128)
        pltpu.make_async_copy(x_h.at[uia.at[pl.ds(0, 128)]], ub.at[hb],
                              sg[b]).wait()
        pltpu.make_async_copy(x_h.at[via.at[pl.ds(0, 128)]], vb.at[hb],
                              sg[b]).wait()

    def compute_store(j, b):
        rbase = b * 128

        def rstep(r0, carry2):
            rows = rbase + r0 * 16 + iota

            def kstep(k, accs):
                alo, ahi = accs
                for jj in range(8):
                    col = (k * 8 + jj + iota) & (DPW - 1)
                    uw = plsc.bitcast(plsc.load_gather(ub, [rows, col]),
                                      jnp.int32)
                    vw = plsc.bitcast(plsc.load_gather(vb, [rows, col]),
                                      jnp.int32)
                    alo = alo + (plsc.bitcast(uw << 16, jnp.float32)
                                 * plsc.bitcast(vw << 16, jnp.float32))
                    ahi = ahi + (plsc.bitcast(uw & himask, jnp.float32)
                                 * plsc.bitcast(vw & himask, jnp.float32))
                return (alo, ahi)

            alo, ahi = lax.fori_loop(0, DPW // 8, kstep, (zero16, zero16))
            ob[pl.ds(r0 * 16, 16)] = alo + ahi
            return carry2

        lax.fori_loop(0, 8, rstep, 0)
        pltpu.sync_copy(ob, out.at[pl.ds(wbase + off_of(j), 128)])

    for b in range(4):
        issue(b, b)

    def group(g, carry):
        for b in range(4):
            j = g * 4 + b
            drain(b)
            compute_store(j, b)
            issue(j + 4, b)
        return carry

    lax.fori_loop(0, (I_NCH - 4) // 4, group, 0)
    for b in range(4):
        j = I_NCH - 4 + b
        drain(b)
        compute_store(j, b)


def _edge_dot(x, u, v):
    f32 = jnp.float32
    k = pl.kernel(
        _edge_dot_body,
        out_type=jax.ShapeDtypeStruct((E,), f32),
        mesh=_sc_mesh(),
        compiler_params=_SC_PARAMS,
        scratch_types=[
            pltpu.VMEM((I_PER_W,), jnp.int32),
            pltpu.VMEM((I_PER_W,), jnp.int32),
            pltpu.VMEM((512, DPW), f32),
            pltpu.VMEM((512, DPW), f32),
            pltpu.VMEM((128,), f32),
            pltpu.SemaphoreType.DMA,
            pltpu.SemaphoreType.DMA,
            pltpu.SemaphoreType.DMA,
            pltpu.SemaphoreType.DMA,
        ],
    )
    return k(x, u, v)


B1 = 2000
NB = N // B1


def _reduce_body(sa_r, small_r, node_r, ss0_r, ss1_r, gw_r, gb_r, lw_r,
                 scale_r, g_r, cs_r, se_r, adj_r):
    i = pl.program_id(0)

    @pl.when(i == 0)
    def _():
        cs_r[...] = jnp.zeros_like(cs_r)
        se_r[...] = jnp.zeros_like(se_r)
        adj_r[...] = jnp.zeros_like(adj_r)

    sa = sa_r[...]
    rf = jnp.concatenate([small_r[...], node_r[...]], axis=1)
    ss = jnp.concatenate([ss0_r[...], ss1_r[...]], axis=1)
    dn = (((0,), (0,)), ((), ()))
    cs_r[...] += jnp.sum(sa, axis=0, keepdims=True)
    se_r[...] += lax.dot_general(sa, rf, dn, preferred_element_type=jnp.float32)
    adj_r[...] += lax.dot_general(sa, ss, dn, preferred_element_type=jnp.float32)

    @pl.when(i == NB - 1)
    def _():
        scale = 1.0 / (jax.nn.relu(cs_r[...] - 1.0) + 1.0)
        scale_c = scale.reshape(64, 1)
        se = se_r[...] * scale_c
        adj = adj_r[...] * scale_c * scale
        support = jnp.dot(se, gw_r[...], preferred_element_type=jnp.float32)
        logits = jnp.dot(adj, support, preferred_element_type=jnp.float32) \
            + gb_r[...]
        fnc = jax.nn.softmax(logits, axis=0)
        fnc_emb = lax.dot_general(fnc, se, dn,
                                  preferred_element_type=jnp.float32)
        m = jnp.dot(fnc, fnc_emb, preferred_element_type=jnp.float32)
        lw = lw_r[...]
        g = jnp.dot(se, lw[HID:2 * HID],
                    preferred_element_type=jnp.float32) \
            + jnp.dot(m, lw[2 * HID:3 * HID],
                      preferred_element_type=jnp.float32)
        scale_r[...] = scale
        g_r[...] = jnp.concatenate(
            [g, jnp.zeros((64, DP2 - DOUT), jnp.float32)], axis=1)


def _reduce_pass(sa, small, node, ss0, ss1, gw, gb, lw):
    f32 = jnp.float32
    return pl.pallas_call(
        _reduce_body,
        grid=(NB,),
        in_specs=[
            pl.BlockSpec((B1, 64), lambda i: (i, 0)),
            pl.BlockSpec((B1, 64), lambda i: (i, 0)),
            pl.BlockSpec((B1, 64), lambda i: (i, 0)),
            pl.BlockSpec((B1, 32), lambda i: (i, 0)),
            pl.BlockSpec((B1, 32), lambda i: (i, 0)),
            pl.BlockSpec((HID, 32), lambda i: (0, 0)),
            pl.BlockSpec((1, 32), lambda i: (0, 0)),
            pl.BlockSpec((3 * HID, DOUT), lambda i: (0, 0)),
        ],
        out_specs=[
            pl.BlockSpec((1, 64), lambda i: (0, 0)),
            pl.BlockSpec((64, DP2), lambda i: (0, 0)),
        ],
        out_shape=[
            jax.ShapeDtypeStruct((1, 64), f32),
            jax.ShapeDtypeStruct((64, DP2), f32),
        ],
        scratch_shapes=[
            pltpu.VMEM((1, 64), f32),
            pltpu.VMEM((64, 128), f32),
            pltpu.VMEM((64, 64), f32),
        ],
    )(sa, small, node, ss0, ss1, gw, gb, lw)


def _x_body(sa_r, small_r, node_r, scale_r, g_r, w1_r, b_r, x_r):
    rf = jnp.concatenate([small_r[...], node_r[...]], axis=1)
    sa = sa_r[...] * scale_r[...]
    x = (
        jnp.dot(rf, w1_r[...], preferred_element_type=jnp.float32)
        + jnp.dot(sa, g_r[...], preferred_element_type=jnp.float32)
        + b_r[...]
    )
    xb = x.astype(jnp.bfloat16)
    lo = lax.bitcast_convert_type(xb[:, :DPW], jnp.uint16).astype(jnp.uint32)
    hi = lax.bitcast_convert_type(xb[:, DPW:], jnp.uint16).astype(jnp.uint32)
    x_r[...] = lax.bitcast_convert_type(lo | (hi << 16), jnp.float32)


def _x_pass(sa, small, node, scale, g, w1p, bp):
    f32 = jnp.float32
    return pl.pallas_call(
        _x_body,
        grid=(NB,),
        in_specs=[
            pl.BlockSpec((B1, 64), lambda i: (i, 0)),
            pl.BlockSpec((B1, 64), lambda i: (i, 0)),
            pl.BlockSpec((B1, 64), lambda i: (i, 0)),
            pl.BlockSpec((1, 64), lambda i: (0, 0)),
            pl.BlockSpec((64, DP2), lambda i: (0, 0)),
            pl.BlockSpec((128, DP2), lambda i: (0, 0)),
            pl.BlockSpec((1, DP2), lambda i: (0, 0)),
        ],
        out_specs=pl.BlockSpec((B1, DPW), lambda i: (i, 0)),
        out_shape=jax.ShapeDtypeStruct((N, DPW), f32),
    )(sa, small, node, scale, g, w1p, bp)


def kernel(lane_feature, type_feature, length_feature, node_feature,
           raw_adj_indices, t_adj, struct_assign, s_edge,
           lane_emb_table, type_emb_table, length_emb_table, node_emb_table,
           gcn_weight, gcn_bias, linear_weight, linear_bias):
    i32 = jnp.int32
    f32 = jnp.float32

    def pad_to(a, n, val):
        return jnp.concatenate([a, jnp.full((n - a.shape[0],), val, a.dtype)])

    lane_i = pad_to(lane_feature.astype(i32), N_PAD, 0)
    type_i = pad_to(type_feature.astype(i32), N_PAD, 0)
    len_i = pad_to(length_feature.astype(i32), N_PAD, 0)
    node_i = pad_to(node_feature.astype(i32), N_PAD, 0)

    small_t = jnp.concatenate([
        jnp.repeat(lane_emb_table, 2000, axis=0),
        jnp.tile(jnp.repeat(type_emb_table, 100, axis=0), (10, 1)),
        jnp.tile(length_emb_table, (200, 1)),
    ], axis=1)

    raw_small, raw_node = _emb_gather(lane_i, type_i, len_i, node_i,
                                      small_t, node_emb_table)

    src = pad_to(raw_adj_indices[0].astype(i32), E_PAD, SENTINEL)
    dst = pad_to(raw_adj_indices[1].astype(i32), E_PAD, 0)
    sa2 = struct_assign.reshape(2 * N, 32)
    zeros_h = jnp.zeros((128, 32), f32)
    ss0, ss1 = _segsum(sa2, src.reshape(D_NCH, 128), dst.reshape(D_NCH, 128),
                       zeros_h)

    gb2 = gcn_bias.reshape(1, 32)
    scale, g = _reduce_pass(
        struct_assign, raw_small[:N], raw_node[:N], ss0[:N], ss1[:N],
        gcn_weight, gb2, linear_weight)

    w1p = jnp.concatenate(
        [linear_weight[:HID], jnp.zeros((HID, DP2 - DOUT), f32)], axis=1)
    bp = jnp.concatenate(
        [linear_bias, jnp.zeros((DP2 - DOUT,), f32)]).reshape(1, DP2)
    x = _x_pass(struct_assign, raw_small[:N], raw_node[:N], scale, g, w1p, bp)

    pred = _edge_dot(x, s_edge[0].astype(i32), s_edge[1].astype(i32))
    return pred


# speedup vs baseline: 1.1697x; 1.1697x over previous
import functools

import jax
import jax.numpy as jnp
from jax import lax
from jax.experimental import pallas as pl
from jax.experimental.pallas import tpu as pltpu
from jax.experimental.pallas import tpu_sc as plsc

N = 50000
E = 800000
C = 64
HID = 128
DOUT = 100
DP = 112
DP2 = 128
DPW = 64

NC = 2
NS = 16
NW = NC * NS

A_CH = 128
A_PER_W = 1792
N_PAD = A_PER_W * NW
A_NCH = A_PER_W // A_CH

E_PAD = 802816
D_NCH = E_PAD // 128
D_PER_T = D_NCH // NS
D_IB = 56
N_ACC = 51200
ACC_PER_T = N_ACC // NS
SENTINEL = N

I_PER_W = E // NW
I_NCH = -(-I_PER_W // 128)


def _sc_mesh():
    return plsc.VectorSubcoreMesh(core_axis_name="c", subcore_axis_name="s")


_SC_PARAMS = pltpu.CompilerParams(use_tc_tiling_on_sc=False,
                                 needs_layout_passes=False)


def _emb_gather_body(lane_i, type_i, len_i, node_i,
                     small_t, node_t,
                     o_small, o_node,
                     ilane, itype, ilen, inode, ismall,
                     bsmall, bnode, sg0, sg1, sw0, sw1):
    wid = lax.axis_index("s") * NC + lax.axis_index("c")
    base = wid * A_PER_W
    pltpu.sync_copy(lane_i.at[pl.ds(base, A_PER_W)], ilane)
    pltpu.sync_copy(type_i.at[pl.ds(base, A_PER_W)], itype)
    pltpu.sync_copy(len_i.at[pl.ds(base, A_PER_W)], ilen)
    pltpu.sync_copy(node_i.at[pl.ds(base, A_PER_W)], inode)

    def fuse(i, carry):
        sl = pl.ds(i * 16, 16)
        ismall[sl] = (ilane[sl] * 20 + itype[sl]) * 100 + ilen[sl]
        return carry

    lax.fori_loop(0, A_PER_W // 16, fuse, 0)
    sg = (sg0, sg1)
    sw = (sw0, sw1)

    def issue(j, b):
        off = j * A_CH
        hb = pl.ds(b * A_CH, A_CH)
        pltpu.async_copy(small_t.at[ismall.at[pl.ds(off, A_CH)]],
                         bsmall.at[hb], sg[b])
        pltpu.async_copy(node_t.at[inode.at[pl.ds(off, A_CH)]],
                         bnode.at[hb], sg[b])

    def drain_g(b):
        hb = pl.ds(b * A_CH, A_CH)
        pltpu.make_async_copy(small_t.at[ismall.at[pl.ds(0, A_CH)]],
                              bsmall.at[hb], sg[b]).wait()
        pltpu.make_async_copy(node_t.at[inode.at[pl.ds(0, A_CH)]],
                              bnode.at[hb], sg[b]).wait()

    def put(j, b):
        gbase = base + j * A_CH
        hb = pl.ds(b * A_CH, A_CH)
        pltpu.async_copy(bsmall.at[hb], o_small.at[pl.ds(gbase, A_CH)], sw[b])
        pltpu.async_copy(bnode.at[hb], o_node.at[pl.ds(gbase, A_CH)], sw[b])

    def drain_w(b):
        hb = pl.ds(b * A_CH, A_CH)
        pltpu.make_async_copy(bsmall.at[hb], o_small.at[pl.ds(0, A_CH)],
                              sw[b]).wait()
        pltpu.make_async_copy(bnode.at[hb], o_node.at[pl.ds(0, A_CH)],
                              sw[b]).wait()

    issue(0, 0)
    issue(1, 1)

    def group(g, carry):
        for b in range(2):
            j = g * 2 + b
            drain_g(b)
            put(j, b)
            drain_w(b)
            issue(j + 2, b)
        return carry

    lax.fori_loop(0, (A_NCH - 2) // 2, group, 0)
    for b in range(2):
        j = A_NCH - 2 + b
        drain_g(b)
        put(j, b)
        drain_w(b)


def _emb_gather(lane_i, type_i, len_i, node_i, small_t, node_t):
    f32 = jnp.float32
    k = pl.kernel(
        _emb_gather_body,
        out_type=[
            jax.ShapeDtypeStruct((N_PAD, 64), f32),
            jax.ShapeDtypeStruct((N_PAD, 64), f32),
        ],
        mesh=_sc_mesh(),
        compiler_params=_SC_PARAMS,
        scratch_types=[
            pltpu.VMEM((A_PER_W,), jnp.int32),
            pltpu.VMEM((A_PER_W,), jnp.int32),
            pltpu.VMEM((A_PER_W,), jnp.int32),
            pltpu.VMEM((A_PER_W,), jnp.int32),
            pltpu.VMEM((A_PER_W,), jnp.int32),
            pltpu.VMEM((2 * A_CH, 64), f32),
            pltpu.VMEM((2 * A_CH, 64), f32),
            pltpu.SemaphoreType.DMA,
            pltpu.SemaphoreType.DMA,
            pltpu.SemaphoreType.DMA,
            pltpu.SemaphoreType.DMA,
        ],
    )
    return k(lane_i, type_i, len_i, node_i, small_t, node_t)


def _segsum_body(sa2, src2d, dst2d, zeros_h, out0, out1,
                 acc, sidx, didx, gbuf, zbuf, sem, sem2):
    c = lax.axis_index("c")
    s = lax.axis_index("s")
    arow = s * ACC_PER_T
    pltpu.sync_copy(zeros_h, zbuf)

    def zstep(w, carry):
        pltpu.sync_copy(zbuf, acc.at[pl.ds(arow + w * 128, 128)])
        return carry

    lax.fori_loop(0, ACC_PER_T // 128, zstep, 0)
    plsc.subcore_barrier()

    c32 = c.astype(jnp.int32)
    sg = (sem, sem2)

    def issue(jj, b):
        pltpu.async_copy(sa2.at[didx.at[jj]], gbuf.at[pl.ds(b * 128, 128)],
                         sg[b])

    def drain(b):
        pltpu.make_async_copy(sa2.at[didx.at[0]],
                              gbuf.at[pl.ds(b * 128, 128)], sg[b]).wait()

    def block(blk, carry):
        cbase = s * D_PER_T + blk * D_IB
        pltpu.sync_copy(src2d.at[pl.ds(cbase, D_IB)], sidx)
        pltpu.sync_copy(dst2d.at[pl.ds(cbase, D_IB)], didx)

        def tstep(jj, carry2):
            for k in range(8):
                sl = pl.ds(k * 16, 16)
                didx[jj, sl] = didx[jj, sl] * 2 + c32
            return carry2

        lax.fori_loop(0, D_IB, tstep, 0)
        issue(0, 0)
        issue(1, 1)

        def group(g, carry2):
            for b in range(2):
                jj = g * 2 + b
                drain(b)
                pltpu.sync_copy(gbuf.at[pl.ds(b * 128, 128)],
                                acc.at[sidx.at[jj]], add=True)
                issue(jj + 2, b)
            return carry2

        lax.fori_loop(0, (D_IB - 2) // 2, group, 0)
        for b in range(2):
            jj = D_IB - 2 + b
            drain(b)
            pltpu.sync_copy(gbuf.at[pl.ds(b * 128, 128)],
                            acc.at[sidx.at[jj]], add=True)
        return carry

    lax.fori_loop(0, D_PER_T // D_IB, block, 0)
    plsc.subcore_barrier()

    @pl.when(c == 0)
    def _():
        def wstep(w, carry):
            r = arow + w * 128
            pltpu.sync_copy(acc.at[pl.ds(r, 128)], zbuf)
            pltpu.sync_copy(zbuf, out0.at[pl.ds(r, 128)])
            return carry
        lax.fori_loop(0, ACC_PER_T // 128, wstep, 0)

    @pl.when(c == 1)
    def _():
        def wstep(w, carry):
            r = arow + w * 128
            pltpu.sync_copy(acc.at[pl.ds(r, 128)], zbuf)
            pltpu.sync_copy(zbuf, out1.at[pl.ds(r, 128)])
            return carry
        lax.fori_loop(0, ACC_PER_T // 128, wstep, 0)


def _segsum(sa2, src2d, dst2d, zeros_h):
    f32 = jnp.float32
    k = pl.kernel(
        _segsum_body,
        out_type=[
            jax.ShapeDtypeStruct((N_ACC, 32), f32),
            jax.ShapeDtypeStruct((N_ACC, 32), f32),
        ],
        mesh=_sc_mesh(),
        compiler_params=_SC_PARAMS,
        scratch_types=[
            pltpu.VMEM_SHARED((N_ACC, 32), f32),
            pltpu.VMEM((D_IB, 128), jnp.int32),
            pltpu.VMEM((D_IB, 128), jnp.int32),
            pltpu.VMEM((256, 32), f32),
            pltpu.VMEM((128, 32), f32),
            pltpu.SemaphoreType.DMA,
            pltpu.SemaphoreType.DMA,
        ],
    )
    return k(sa2, src2d, dst2d, zeros_h)


def _edge_dot_body(x_h, u_h, v_h, out, uia, via, ub, vb, ob,
                   sg0, sg1, sg2, sg3):
    wid = lax.axis_index("s") * NC + lax.axis_index("c")
    wbase = wid * I_PER_W
    pltpu.sync_copy(u_h.at[pl.ds(wbase, I_PER_W)], uia)
    pltpu.sync_copy(v_h.at[pl.ds(wbase, I_PER_W)], via)
    iota = lax.iota(jnp.int32, 16)
    sg = (sg0, sg1, sg2, sg3)
    himask = jnp.broadcast_to(jnp.int32(-65536), (16,))
    zero16 = jnp.zeros((16,), jnp.float32)

    def off_of(j):
        return jnp.minimum(j * 128, I_PER_W - 128)

    def issue(j, b):
        off = off_of(j)
        hb = pl.ds(b * 128, 128)
        pltpu.async_copy(x_h.at[uia.at[pl.ds(off, 128)]], ub.at[hb], sg[b])
        pltpu.async_copy(x_h.at[via.at[pl.ds(off, 128)]], vb.at[hb], sg[b])

    def drain(b):
        hb = pl.ds(b * 128, 128)
        pltpu.make_async_copy(x_h.at[uia.at[pl.ds(0, 128)]], ub.at[hb],
                              sg[b]).wait()
        pltpu.make_async_copy(x_h.at[via.at[pl.ds(0, 128)]], vb.at[hb],
                              sg[b]).wait()

    def compute_store(j, b):
        rbase = b * 128

        def rstep(r0, carry2):
            rows = rbase + r0 * 16 + iota

            def kstep(k, accs):
                alo, ahi = accs
                for jj in range(8):
                    col = (k * 8 + jj + iota) & (DPW - 1)
                    uw = plsc.bitcast(plsc.load_gather(ub, [rows, col]),
                                      jnp.int32)
                    vw = plsc.bitcast(plsc.load_gather(vb, [rows, col]),
                                      jnp.int32)
                    alo = alo + (plsc.bitcast(uw << 16, jnp.float32)
                                 * plsc.bitcast(vw << 16, jnp.float32))
                    ahi = ahi + (plsc.bitcast(uw & himask, jnp.float32)
                                 * plsc.bitcast(vw & himask, jnp.float32))
                return (alo, ahi)

            alo, ahi = lax.fori_loop(0, DPW // 8, kstep, (zero16, zero16))
            ob[pl.ds(r0 * 16, 16)] = alo + ahi
            return carry2

        lax.fori_loop(0, 8, rstep, 0)
        pltpu.sync_copy(ob, out.at[pl.ds(wbase + off_of(j), 128)])

    for b in range(4):
        issue(b, b)

    def group(g, carry):
        for b in range(4):
            j = g * 4 + b
            drain(b)
            compute_store(j, b)
            issue(j + 4, b)
        return carry

    lax.fori_loop(0, (I_NCH - 4) // 4, group, 0)
    for b in range(4):
        j = I_NCH - 4 + b
        drain(b)
        compute_store(j, b)


def _edge_dot(x, u, v):
    f32 = jnp.float32
    k = pl.kernel(
        _edge_dot_body,
        out_type=jax.ShapeDtypeStruct((E,), f32),
        mesh=_sc_mesh(),
        compiler_params=_SC_PARAMS,
        scratch_types=[
            pltpu.VMEM((I_PER_W,), jnp.int32),
            pltpu.VMEM((I_PER_W,), jnp.int32),
            pltpu.VMEM((512, DPW), f32),
            pltpu.VMEM((512, DPW), f32),
            pltpu.VMEM((128,), f32),
            pltpu.SemaphoreType.DMA,
            pltpu.SemaphoreType.DMA,
            pltpu.SemaphoreType.DMA,
            pltpu.SemaphoreType.DMA,
        ],
    )
    return k(x, u, v)


B1 = 2000
NB = N // B1


def _reduce_body(sa_r, small_r, node_r, ss0_r, ss1_r, gw_r, gb_r, lw_r,
                 scale_r, g_r, cs_r, se_r, adj_r):
    i = pl.program_id(0)

    @pl.when(i == 0)
    def _():
        cs_r[...] = jnp.zeros_like(cs_r)
        se_r[...] = jnp.zeros_like(se_r)
        adj_r[...] = jnp.zeros_like(adj_r)

    sa = sa_r[...]
    rf = jnp.concatenate([small_r[...], node_r[...]], axis=1)
    ss = jnp.concatenate([ss0_r[...], ss1_r[...]], axis=1)
    dn = (((0,), (0,)), ((), ()))
    cs_r[...] += jnp.sum(sa, axis=0, keepdims=True)
    se_r[...] += lax.dot_general(sa, rf, dn, preferred_element_type=jnp.float32)
    adj_r[...] += lax.dot_general(sa, ss, dn, preferred_element_type=jnp.float32)

    @pl.when(i == NB - 1)
    def _():
        scale = 1.0 / (jax.nn.relu(cs_r[...] - 1.0) + 1.0)
        scale_c = scale.reshape(64, 1)
        se = se_r[...] * scale_c
        adj = adj_r[...] * scale_c * scale
        support = jnp.dot(se, gw_r[...], preferred_element_type=jnp.float32)
        logits = jnp.dot(adj, support, preferred_element_type=jnp.float32) \
            + gb_r[...]
        fnc = jax.nn.softmax(logits, axis=0)
        fnc_emb = lax.dot_general(fnc, se, dn,
                                  preferred_element_type=jnp.float32)
        m = jnp.dot(fnc, fnc_emb, preferred_element_type=jnp.float32)
        lw = lw_r[...]
        g = jnp.dot(se, lw[HID:2 * HID],
                    preferred_element_type=jnp.float32) \
            + jnp.dot(m, lw[2 * HID:3 * HID],
                      preferred_element_type=jnp.float32)
        scale_r[...] = scale
        g_r[...] = jnp.concatenate(
            [g, jnp.zeros((64, DP2 - DOUT), jnp.float32)], axis=1)


def _reduce_pass(sa, small, node, ss0, ss1, gw, gb, lw):
    f32 = jnp.float32
    return pl.pallas_call(
        _reduce_body,
        grid=(NB,),
        in_specs=[
            pl.BlockSpec((B1, 64), lambda i: (i, 0)),
            pl.BlockSpec((B1, 64), lambda i: (i, 0)),
            pl.BlockSpec((B1, 64), lambda i: (i, 0)),
            pl.BlockSpec((B1, 32), lambda i: (i, 0)),
            pl.BlockSpec((B1, 32), lambda i: (i, 0)),
            pl.BlockSpec((HID, 32), lambda i: (0, 0)),
            pl.BlockSpec((1, 32), lambda i: (0, 0)),
            pl.BlockSpec((3 * HID, DOUT), lambda i: (0, 0)),
        ],
        out_specs=[
            pl.BlockSpec((1, 64), lambda i: (0, 0)),
            pl.BlockSpec((64, DP2), lambda i: (0, 0)),
        ],
        out_shape=[
            jax.ShapeDtypeStruct((1, 64), f32),
            jax.ShapeDtypeStruct((64, DP2), f32),
        ],
        scratch_shapes=[
            pltpu.VMEM((1, 64), f32),
            pltpu.VMEM((64, 128), f32),
            pltpu.VMEM((64, 64), f32),
        ],
    )(sa, small, node, ss0, ss1, gw, gb, lw)


def _x_body(sa_r, small_r, node_r, scale_r, g_r, w1_r, b_r, x_r):
    rf = jnp.concatenate([small_r[...], node_r[...]], axis=1)
    sa = sa_r[...] * scale_r[...]
    x = (
        jnp.dot(rf, w1_r[...], preferred_element_type=jnp.float32)
        + jnp.dot(sa, g_r[...], preferred_element_type=jnp.float32)
        + b_r[...]
    )
    xb = x.astype(jnp.bfloat16)
    lo = lax.bitcast_convert_type(xb[:, :DPW], jnp.uint16).astype(jnp.uint32)
    hi = lax.bitcast_convert_type(xb[:, DPW:], jnp.uint16).astype(jnp.uint32)
    x_r[...] = lax.bitcast_convert_type(lo | (hi << 16), jnp.float32)


def _x_pass(sa, small, node, scale, g, w1p, bp):
    f32 = jnp.float32
    return pl.pallas_call(
        _x_body,
        grid=(NB,),
        in_specs=[
            pl.BlockSpec((B1, 64), lambda i: (i, 0)),
            pl.BlockSpec((B1, 64), lambda i: (i, 0)),
            pl.BlockSpec((B1, 64), lambda i: (i, 0)),
            pl.BlockSpec((1, 64), lambda i: (0, 0)),
            pl.BlockSpec((64, DP2), lambda i: (0, 0)),
            pl.BlockSpec((128, DP2), lambda i: (0, 0)),
            pl.BlockSpec((1, DP2), lambda i: (0, 0)),
        ],
        out_specs=pl.BlockSpec((B1, DPW), lambda i: (i, 0)),
        out_shape=jax.ShapeDtypeStruct((N, DPW), f32),
    )(sa, small, node, scale, g, w1p, bp)


def kernel(lane_feature, type_feature, length_feature, node_feature,
           raw_adj_indices, t_adj, struct_assign, s_edge,
           lane_emb_table, type_emb_table, length_emb_table, node_emb_table,
           gcn_weight, gcn_bias, linear_weight, linear_bias):
    i32 = jnp.int32
    f32 = jnp.float32

    def pad_to(a, n, val):
        return jnp.concatenate([a, jnp.full((n - a.shape[0],), val, a.dtype)])

    lane_i = pad_to(lane_feature.astype(i32), N_PAD, 0)
    type_i = pad_to(type_feature.astype(i32), N_PAD, 0)
    len_i = pad_to(length_feature.astype(i32), N_PAD, 0)
    node_i = pad_to(node_feature.astype(i32), N_PAD, 0)

    small_t = jnp.concatenate([
        jnp.repeat(lane_emb_table, 2000, axis=0),
        jnp.tile(jnp.repeat(type_emb_table, 100, axis=0), (10, 1)),
        jnp.tile(length_emb_table, (200, 1)),
    ], axis=1)

    raw_small, raw_node = _emb_gather(lane_i, type_i, len_i, node_i,
                                      small_t, node_emb_table)

    src = pad_to(raw_adj_indices[0].astype(i32), E_PAD, SENTINEL)
    dst = pad_to(raw_adj_indices[1].astype(i32), E_PAD, 0)
    sa2 = struct_assign.reshape(2 * N, 32)
    zeros_h = jnp.zeros((128, 32), f32)
    ss0, ss1 = _segsum(sa2, src.reshape(D_NCH, 128), dst.reshape(D_NCH, 128),
                       zeros_h)

    gb2 = gcn_bias.reshape(1, 32)
    scale, g = _reduce_pass(
        struct_assign, raw_small[:N], raw_node[:N], ss0[:N], ss1[:N],
        gcn_weight, gb2, linear_weight)

    w1p = jnp.concatenate(
        [linear_weight[:HID], jnp.zeros((HID, DP2 - DOUT), f32)], axis=1)
    bp = jnp.concatenate(
        [linear_bias, jnp.zeros((DP2 - DOUT,), f32)]).reshape(1, DP2)
    x = _x_pass(struct_assign, raw_small[:N], raw_node[:N], scale, g, w1p, bp)

    pred = _edge_dot(x, s_edge[0].astype(i32), s_edge[1].astype(i32))
    return pred


# speedup vs baseline: 1.2398x; 1.0599x over previous
import functools

import jax
import jax.numpy as jnp
from jax import lax
from jax.experimental import pallas as pl
from jax.experimental.pallas import tpu as pltpu
from jax.experimental.pallas import tpu_sc as plsc

N = 50000
E = 800000
C = 64
HID = 128
DOUT = 100
DP = 112
DP2 = 128
DPW = 64

NC = 2
NS = 16
NW = NC * NS

A_CH = 128
A_PER_W = 1792
N_PAD = A_PER_W * NW
A_NCH = A_PER_W // A_CH

E_PAD = 802816
D_NCH = E_PAD // 128
D_PER_T = D_NCH // NS
D_IB = 28
N_ACC = 51200
ACC_PER_T = N_ACC // NS
SENTINEL = N

I_PER_W = E // NW
I_NCH = -(-I_PER_W // 128)


def _sc_mesh():
    return plsc.VectorSubcoreMesh(core_axis_name="c", subcore_axis_name="s")


_SC_PARAMS = pltpu.CompilerParams(use_tc_tiling_on_sc=False,
                                 needs_layout_passes=False)


def _emb_gather_body(lane_i, type_i, len_i, node_i,
                     small_t, node_t,
                     o_small, o_node,
                     ilane, itype, ilen, inode, ismall,
                     bsmall, bnode, sg0, sg1, sw0, sw1):
    wid = lax.axis_index("s") * NC + lax.axis_index("c")
    base = wid * A_PER_W
    pltpu.sync_copy(lane_i.at[pl.ds(base, A_PER_W)], ilane)
    pltpu.sync_copy(type_i.at[pl.ds(base, A_PER_W)], itype)
    pltpu.sync_copy(len_i.at[pl.ds(base, A_PER_W)], ilen)
    pltpu.sync_copy(node_i.at[pl.ds(base, A_PER_W)], inode)

    def fuse(i, carry):
        sl = pl.ds(i * 16, 16)
        ismall[sl] = (ilane[sl] * 20 + itype[sl]) * 100 + ilen[sl]
        return carry

    lax.fori_loop(0, A_PER_W // 16, fuse, 0)
    sg = (sg0, sg1)
    sw = (sw0, sw1)

    def issue(j, b):
        off = j * A_CH
        hb = pl.ds(b * A_CH, A_CH)
        pltpu.async_copy(small_t.at[ismall.at[pl.ds(off, A_CH)]],
                         bsmall.at[hb], sg[b])
        pltpu.async_copy(node_t.at[inode.at[pl.ds(off, A_CH)]],
                         bnode.at[hb], sg[b])

    def drain_g(b):
        hb = pl.ds(b * A_CH, A_CH)
        pltpu.make_async_copy(small_t.at[ismall.at[pl.ds(0, A_CH)]],
                              bsmall.at[hb], sg[b]).wait()
        pltpu.make_async_copy(node_t.at[inode.at[pl.ds(0, A_CH)]],
                              bnode.at[hb], sg[b]).wait()

    def put(j, b):
        gbase = base + j * A_CH
        hb = pl.ds(b * A_CH, A_CH)
        pltpu.async_copy(bsmall.at[hb], o_small.at[pl.ds(gbase, A_CH)], sw[b])
        pltpu.async_copy(bnode.at[hb], o_node.at[pl.ds(gbase, A_CH)], sw[b])

    def drain_w(b):
        hb = pl.ds(b * A_CH, A_CH)
        pltpu.make_async_copy(bsmall.at[hb], o_small.at[pl.ds(0, A_CH)],
                              sw[b]).wait()
        pltpu.make_async_copy(bnode.at[hb], o_node.at[pl.ds(0, A_CH)],
                              sw[b]).wait()

    issue(0, 0)
    issue(1, 1)

    def group(g, carry):
        for b in range(2):
            j = g * 2 + b
            drain_g(b)
            put(j, b)
            drain_w(b)
            issue(j + 2, b)
        return carry

    lax.fori_loop(0, (A_NCH - 2) // 2, group, 0)
    for b in range(2):
        j = A_NCH - 2 + b
        drain_g(b)
        put(j, b)
        drain_w(b)


def _emb_gather(lane_i, type_i, len_i, node_i, small_t, node_t):
    f32 = jnp.float32
    k = pl.kernel(
        _emb_gather_body,
        out_type=[
            jax.ShapeDtypeStruct((N_PAD, 64), f32),
            jax.ShapeDtypeStruct((N_PAD, 64), f32),
        ],
        mesh=_sc_mesh(),
        compiler_params=_SC_PARAMS,
        scratch_types=[
            pltpu.VMEM((A_PER_W,), jnp.int32),
            pltpu.VMEM((A_PER_W,), jnp.int32),
            pltpu.VMEM((A_PER_W,), jnp.int32),
            pltpu.VMEM((A_PER_W,), jnp.int32),
            pltpu.VMEM((A_PER_W,), jnp.int32),
            pltpu.VMEM((2 * A_CH, 64), f32),
            pltpu.VMEM((2 * A_CH, 64), f32),
            pltpu.SemaphoreType.DMA,
            pltpu.SemaphoreType.DMA,
            pltpu.SemaphoreType.DMA,
            pltpu.SemaphoreType.DMA,
        ],
    )
    return k(lane_i, type_i, len_i, node_i, small_t, node_t)


def _segsum_body(sa2, src2d, dst2d, zeros_h, out0, out1,
                 acc, sidx, didx, gbuf, zbuf, sem, sem2, sem3, sem4):
    c = lax.axis_index("c")
    s = lax.axis_index("s")
    arow = s * ACC_PER_T
    pltpu.sync_copy(zeros_h, zbuf)

    def zstep(w, carry):
        pltpu.sync_copy(zbuf, acc.at[pl.ds(arow + w * 128, 128)])
        return carry

    lax.fori_loop(0, ACC_PER_T // 128, zstep, 0)
    plsc.subcore_barrier()

    c32 = c.astype(jnp.int32)
    sg = (sem, sem2, sem3, sem4)

    def issue(jj, b):
        pltpu.async_copy(sa2.at[didx.at[jj]], gbuf.at[pl.ds(b * 128, 128)],
                         sg[b])

    def drain(b):
        pltpu.make_async_copy(sa2.at[didx.at[0]],
                              gbuf.at[pl.ds(b * 128, 128)], sg[b]).wait()

    def block(blk, carry):
        cbase = s * D_PER_T + blk * D_IB
        pltpu.sync_copy(src2d.at[pl.ds(cbase, D_IB)], sidx)
        pltpu.sync_copy(dst2d.at[pl.ds(cbase, D_IB)], didx)

        def tstep(jj, carry2):
            for k in range(8):
                sl = pl.ds(k * 16, 16)
                didx[jj, sl] = didx[jj, sl] * 2 + c32
            return carry2

        lax.fori_loop(0, D_IB, tstep, 0)
        for b in range(4):
            issue(b, b)

        def group(g, carry2):
            for b in range(4):
                jj = g * 4 + b
                drain(b)
                pltpu.sync_copy(gbuf.at[pl.ds(b * 128, 128)],
                                acc.at[sidx.at[jj]], add=True)
                issue(jj + 4, b)
            return carry2

        lax.fori_loop(0, (D_IB - 4) // 4, group, 0)
        for b in range(4):
            jj = D_IB - 4 + b
            drain(b)
            pltpu.sync_copy(gbuf.at[pl.ds(b * 128, 128)],
                            acc.at[sidx.at[jj]], add=True)
        return carry

    lax.fori_loop(0, D_PER_T // D_IB, block, 0)
    plsc.subcore_barrier()

    @pl.when(c == 0)
    def _():
        def wstep(w, carry):
            r = arow + w * 128
            pltpu.sync_copy(acc.at[pl.ds(r, 128)], zbuf)
            pltpu.sync_copy(zbuf, out0.at[pl.ds(r, 128)])
            return carry
        lax.fori_loop(0, ACC_PER_T // 128, wstep, 0)

    @pl.when(c == 1)
    def _():
        def wstep(w, carry):
            r = arow + w * 128
            pltpu.sync_copy(acc.at[pl.ds(r, 128)], zbuf)
            pltpu.sync_copy(zbuf, out1.at[pl.ds(r, 128)])
            return carry
        lax.fori_loop(0, ACC_PER_T // 128, wstep, 0)


def _segsum(sa2, src2d, dst2d, zeros_h):
    f32 = jnp.float32
    k = pl.kernel(
        _segsum_body,
        out_type=[
            jax.ShapeDtypeStruct((N_ACC, 32), f32),
            jax.ShapeDtypeStruct((N_ACC, 32), f32),
        ],
        mesh=_sc_mesh(),
        compiler_params=_SC_PARAMS,
        scratch_types=[
            pltpu.VMEM_SHARED((N_ACC, 32), f32),
            pltpu.VMEM((D_IB, 128), jnp.int32),
            pltpu.VMEM((D_IB, 128), jnp.int32),
            pltpu.VMEM((512, 32), f32),
            pltpu.VMEM((128, 32), f32),
            pltpu.SemaphoreType.DMA,
            pltpu.SemaphoreType.DMA,
            pltpu.SemaphoreType.DMA,
            pltpu.SemaphoreType.DMA,
        ],
    )
    return k(sa2, src2d, dst2d, zeros_h)


def _edge_dot_body(x_h, u_h, v_h, out, uia, via, ub, vb, ob,
                   sg0, sg1, sg2, sg3):
    wid = lax.axis_index("s") * NC + lax.axis_index("c")
    wbase = wid * I_PER_W
    pltpu.sync_copy(u_h.at[pl.ds(wbase, I_PER_W)], uia)
    pltpu.sync_copy(v_h.at[pl.ds(wbase, I_PER_W)], via)
    iota = lax.iota(jnp.int32, 16)
    sg = (sg0, sg1, sg2, sg3)
    himask = jnp.broadcast_to(jnp.int32(-65536), (16,))
    zero16 = jnp.zeros((16,), jnp.float32)

    def off_of(j):
        return jnp.minimum(j * 128, I_PER_W - 128)

    def issue(j, b):
        off = off_of(j)
        hb = pl.ds(b * 128, 128)
        pltpu.async_copy(x_h.at[uia.at[pl.ds(off, 128)]], ub.at[hb], sg[b])
        pltpu.async_copy(x_h.at[via.at[pl.ds(off, 128)]], vb.at[hb], sg[b])

    def drain(b):
        hb = pl.ds(b * 128, 128)
        pltpu.make_async_copy(x_h.at[uia.at[pl.ds(0, 128)]], ub.at[hb],
                              sg[b]).wait()
        pltpu.make_async_copy(x_h.at[via.at[pl.ds(0, 128)]], vb.at[hb],
                              sg[b]).wait()

    def compute_store(j, b):
        rbase = b * 128

        def rstep(r0, carry2):
            rows = rbase + r0 * 16 + iota

            def kstep(k, accs):
                alo, ahi = accs
                for jj in range(8):
                    col = (k * 8 + jj + iota) & (DPW - 1)
                    uw = plsc.bitcast(plsc.load_gather(ub, [rows, col]),
                                      jnp.int32)
                    vw = plsc.bitcast(plsc.load_gather(vb, [rows, col]),
                                      jnp.int32)
                    alo = alo + (plsc.bitcast(uw << 16, jnp.float32)
                                 * plsc.bitcast(vw << 16, jnp.float32))
                    ahi = ahi + (plsc.bitcast(uw & himask, jnp.float32)
                                 * plsc.bitcast(vw & himask, jnp.float32))
                return (alo, ahi)

            alo, ahi = lax.fori_loop(0, DPW // 8, kstep, (zero16, zero16))
            ob[pl.ds(r0 * 16, 16)] = alo + ahi
            return carry2

        lax.fori_loop(0, 8, rstep, 0)
        pltpu.sync_copy(ob, out.at[pl.ds(wbase + off_of(j), 128)])

    for b in range(4):
        issue(b, b)

    def group(g, carry):
        for b in range(4):
            j = g * 4 + b
            drain(b)
            compute_store(j, b)
            issue(j + 4, b)
        return carry

    lax.fori_loop(0, (I_NCH - 4) // 4, group, 0)
    for b in range(4):
        j = I_NCH - 4 + b
        drain(b)
        compute_store(j, b)


def _edge_dot(x, u, v):
    f32 = jnp.float32
    k = pl.kernel(
        _edge_dot_body,
        out_type=jax.ShapeDtypeStruct((E,), f32),
        mesh=_sc_mesh(),
        compiler_params=_SC_PARAMS,
        scratch_types=[
            pltpu.VMEM((I_PER_W,), jnp.int32),
            pltpu.VMEM((I_PER_W,), jnp.int32),
            pltpu.VMEM((512, DPW), f32),
            pltpu.VMEM((512, DPW), f32),
            pltpu.VMEM((128,), f32),
            pltpu.SemaphoreType.DMA,
            pltpu.SemaphoreType.DMA,
            pltpu.SemaphoreType.DMA,
            pltpu.SemaphoreType.DMA,
        ],
    )
    return k(x, u, v)


B1 = 2000
NB = N // B1


def _reduce_body(sa_r, small_r, node_r, ss0_r, ss1_r, gw_r, gb_r, lw_r,
                 scale_r, g_r, cs_r, se_r, adj_r):
    i = pl.program_id(0)

    @pl.when(i == 0)
    def _():
        cs_r[...] = jnp.zeros_like(cs_r)
        se_r[...] = jnp.zeros_like(se_r)
        adj_r[...] = jnp.zeros_like(adj_r)

    sa = sa_r[...]
    rf = jnp.concatenate([small_r[...], node_r[...]], axis=1)
    ss = jnp.concatenate([ss0_r[...], ss1_r[...]], axis=1)
    dn = (((0,), (0,)), ((), ()))
    cs_r[...] += jnp.sum(sa, axis=0, keepdims=True)
    se_r[...] += lax.dot_general(sa, rf, dn, preferred_element_type=jnp.float32)
    adj_r[...] += lax.dot_general(sa, ss, dn, preferred_element_type=jnp.float32)

    @pl.when(i == NB - 1)
    def _():
        scale = 1.0 / (jax.nn.relu(cs_r[...] - 1.0) + 1.0)
        scale_c = scale.reshape(64, 1)
        se = se_r[...] * scale_c
        adj = adj_r[...] * scale_c * scale
        support = jnp.dot(se, gw_r[...], preferred_element_type=jnp.float32)
        logits = jnp.dot(adj, support, preferred_element_type=jnp.float32) \
            + gb_r[...]
        fnc = jax.nn.softmax(logits, axis=0)
        fnc_emb = lax.dot_general(fnc, se, dn,
                                  preferred_element_type=jnp.float32)
        m = jnp.dot(fnc, fnc_emb, preferred_element_type=jnp.float32)
        lw = lw_r[...]
        g = jnp.dot(se, lw[HID:2 * HID],
                    preferred_element_type=jnp.float32) \
            + jnp.dot(m, lw[2 * HID:3 * HID],
                      preferred_element_type=jnp.float32)
        scale_r[...] = scale
        g_r[...] = jnp.concatenate(
            [g, jnp.zeros((64, DP2 - DOUT), jnp.float32)], axis=1)


def _reduce_pass(sa, small, node, ss0, ss1, gw, gb, lw):
    f32 = jnp.float32
    return pl.pallas_call(
        _reduce_body,
        grid=(NB,),
        in_specs=[
            pl.BlockSpec((B1, 64), lambda i: (i, 0)),
            pl.BlockSpec((B1, 64), lambda i: (i, 0)),
            pl.BlockSpec((B1, 64), lambda i: (i, 0)),
            pl.BlockSpec((B1, 32), lambda i: (i, 0)),
            pl.BlockSpec((B1, 32), lambda i: (i, 0)),
            pl.BlockSpec((HID, 32), lambda i: (0, 0)),
            pl.BlockSpec((1, 32), lambda i: (0, 0)),
            pl.BlockSpec((3 * HID, DOUT), lambda i: (0, 0)),
        ],
        out_specs=[
            pl.BlockSpec((1, 64), lambda i: (0, 0)),
            pl.BlockSpec((64, DP2), lambda i: (0, 0)),
        ],
        out_shape=[
            jax.ShapeDtypeStruct((1, 64), f32),
            jax.ShapeDtypeStruct((64, DP2), f32),
        ],
        scratch_shapes=[
            pltpu.VMEM((1, 64), f32),
            pltpu.VMEM((64, 128), f32),
            pltpu.VMEM((64, 64), f32),
        ],
    )(sa, small, node, ss0, ss1, gw, gb, lw)


def _x_body(sa_r, small_r, node_r, scale_r, g_r, w1_r, b_r, x_r):
    rf = jnp.concatenate([small_r[...], node_r[...]], axis=1)
    sa = sa_r[...] * scale_r[...]
    x = (
        jnp.dot(rf, w1_r[...], preferred_element_type=jnp.float32)
        + jnp.dot(sa, g_r[...], preferred_element_type=jnp.float32)
        + b_r[...]
    )
    xb = x.astype(jnp.bfloat16)
    lo = lax.bitcast_convert_type(xb[:, :DPW], jnp.uint16).astype(jnp.uint32)
    hi = lax.bitcast_convert_type(xb[:, DPW:], jnp.uint16).astype(jnp.uint32)
    x_r[...] = lax.bitcast_convert_type(lo | (hi << 16), jnp.float32)


def _x_pass(sa, small, node, scale, g, w1p, bp):
    f32 = jnp.float32
    return pl.pallas_call(
        _x_body,
        grid=(NB,),
        in_specs=[
            pl.BlockSpec((B1, 64), lambda i: (i, 0)),
            pl.BlockSpec((B1, 64), lambda i: (i, 0)),
            pl.BlockSpec((B1, 64), lambda i: (i, 0)),
            pl.BlockSpec((1, 64), lambda i: (0, 0)),
            pl.BlockSpec((64, DP2), lambda i: (0, 0)),
            pl.BlockSpec((128, DP2), lambda i: (0, 0)),
            pl.BlockSpec((1, DP2), lambda i: (0, 0)),
        ],
        out_specs=pl.BlockSpec((B1, DPW), lambda i: (i, 0)),
        out_shape=jax.ShapeDtypeStruct((N, DPW), f32),
    )(sa, small, node, scale, g, w1p, bp)


def kernel(lane_feature, type_feature, length_feature, node_feature,
           raw_adj_indices, t_adj, struct_assign, s_edge,
           lane_emb_table, type_emb_table, length_emb_table, node_emb_table,
           gcn_weight, gcn_bias, linear_weight, linear_bias):
    i32 = jnp.int32
    f32 = jnp.float32

    def pad_to(a, n, val):
        return jnp.concatenate([a, jnp.full((n - a.shape[0],), val, a.dtype)])

    lane_i = pad_to(lane_feature.astype(i32), N_PAD, 0)
    type_i = pad_to(type_feature.astype(i32), N_PAD, 0)
    len_i = pad_to(length_feature.astype(i32), N_PAD, 0)
    node_i = pad_to(node_feature.astype(i32), N_PAD, 0)

    small_t = jnp.concatenate([
        jnp.repeat(lane_emb_table, 2000, axis=0),
        jnp.tile(jnp.repeat(type_emb_table, 100, axis=0), (10, 1)),
        jnp.tile(length_emb_table, (200, 1)),
    ], axis=1)

    raw_small, raw_node = _emb_gather(lane_i, type_i, len_i, node_i,
                                      small_t, node_emb_table)

    src = pad_to(raw_adj_indices[0].astype(i32), E_PAD, SENTINEL)
    dst = pad_to(raw_adj_indices[1].astype(i32), E_PAD, 0)
    sa2 = struct_assign.reshape(2 * N, 32)
    zeros_h = jnp.zeros((128, 32), f32)
    ss0, ss1 = _segsum(sa2, src.reshape(D_NCH, 128), dst.reshape(D_NCH, 128),
                       zeros_h)

    gb2 = gcn_bias.reshape(1, 32)
    scale, g = _reduce_pass(
        struct_assign, raw_small[:N], raw_node[:N], ss0[:N], ss1[:N],
        gcn_weight, gb2, linear_weight)

    w1p = jnp.concatenate(
        [linear_weight[:HID], jnp.zeros((HID, DP2 - DOUT), f32)], axis=1)
    bp = jnp.concatenate(
        [linear_bias, jnp.zeros((DP2 - DOUT,), f32)]).reshape(1, DP2)
    x = _x_pass(struct_assign, raw_small[:N], raw_node[:N], scale, g, w1p, bp)

    pred = _edge_dot(x, s_edge[0].astype(i32), s_edge[1].astype(i32))
    return pred
